# Initial kernel scaffold; baseline (speedup 1.0000x reference)
import jax
import jax.numpy as jnp
from jax.experimental import pallas as pl


def kernel(indices, table):
    raise NotImplementedError("write your pallas kernel here")


# speedup vs baseline: 2.9337x; 2.9337x over previous
import functools
import math

import jax
import jax.numpy as jnp
from jax import lax
from jax.experimental import pallas as pl
from jax.experimental.pallas import tpu as pltpu
from jax.experimental.pallas import tpu_sc as plsc

NUM_EMBEDDINGS = 100000
D = 128
LANES = 16
NC = 2
NS = 16
NW = NC * NS
CHUNK = 128
NBUF = 5
SCALE = math.sqrt(float(D))


def _build(B):
    per_w = B // NW
    n_chunks = per_w // CHUNK
    assert per_w * NW == B and n_chunks * CHUNK == per_w
    assert n_chunks % NBUF == 0

    mesh = plsc.VectorSubcoreMesh(core_axis_name="c", subcore_axis_name="s")

    @functools.partial(
        pl.kernel,
        out_type=jax.ShapeDtypeStruct((B, D), jnp.float32),
        mesh=mesh,
        scratch_types=[
            pltpu.VMEM((n_chunks, CHUNK), jnp.int32),
            [pltpu.VMEM((CHUNK, D), jnp.float32) for _ in range(NBUF)],
            [pltpu.SemaphoreType.DMA for _ in range(NBUF)],
            [pltpu.SemaphoreType.DMA for _ in range(NBUF)],
        ],
    )
    def emb_kernel(idx_hbm, table_hbm, out_hbm, idx_v, bufs, gsems, ssems):
        wid = lax.axis_index("s") * NC + lax.axis_index("c")
        row0 = wid * per_w

        pltpu.sync_copy(idx_hbm.at[wid], idx_v)

        def fire_gather(g, b):
            pltpu.async_copy(table_hbm.at[idx_v.at[g]], bufs[b], gsems[b])

        def wait_gather(g, b):
            pltpu.make_async_copy(
                table_hbm.at[idx_v.at[g]], bufs[b], gsems[b]).wait()

        def out_slice(g):
            return out_hbm.at[pl.ds(row0 + g * CHUNK, CHUNK)]

        def fire_scatter(g, b):
            pltpu.async_copy(bufs[b], out_slice(g), ssems[b])

        def wait_scatter(g, b):
            pltpu.make_async_copy(bufs[b], out_slice(g), ssems[b]).wait()

        def scale_buf(b):
            buf = bufs[b]

            def row(r, carry):
                for c in range(D // LANES):
                    sl = (r, pl.ds(c * LANES, LANES))
                    buf[sl] = buf[sl] * SCALE
                return carry

            lax.fori_loop(0, CHUNK, row, 0)

        for b in range(NBUF):
            fire_gather(b, b)

        def body(go, refire):
            for b in range(NBUF):
                g = go + b
                wait_gather(g, b)
                scale_buf(b)
                fire_scatter(g, b)
            if refire:
                for b in range(NBUF):
                    g = go + b
                    wait_scatter(g, b)
                    fire_gather(g + NBUF, b)

        @pl.loop(0, (n_chunks // NBUF - 1) * NBUF, step=NBUF)
        def _(go):
            body(go, refire=True)

        last = (n_chunks // NBUF - 1) * NBUF
        body(last, refire=False)
        for b in range(NBUF):
            wait_scatter(last + b, b)

    return emb_kernel


def kernel(indices, table):
    B = indices.size
    idx3 = indices.reshape(NW, B // (NW * CHUNK), CHUNK)
    out = _build(B)(idx3, table)
    return out.reshape(indices.shape + (D,))


# speedup vs baseline: 5.2153x; 1.7777x over previous
import functools
import math

import jax
import jax.numpy as jnp
from jax import lax
from jax.experimental import pallas as pl
from jax.experimental.pallas import tpu as pltpu
from jax.experimental.pallas import tpu_sc as plsc

D = 128
LANES = 16
NC = 2
NS = 16
NW = NC * NS
NBUF = 8


def _build(batch, hist):
    per_w = batch // NW
    assert per_w * NW == batch and per_w % NBUF == 0
    scale = math.sqrt(float(D))

    mesh = plsc.VectorSubcoreMesh(core_axis_name="c", subcore_axis_name="s")

    @functools.partial(
        pl.kernel,
        out_type=jax.ShapeDtypeStruct((batch, hist, D), jnp.float32),
        mesh=mesh,
        scratch_types=[
            pltpu.VMEM((per_w, hist), jnp.int32),
            [pltpu.VMEM((hist, D), jnp.float32) for _ in range(NBUF)],
            [pltpu.SemaphoreType.DMA for _ in range(NBUF)],
            [pltpu.SemaphoreType.DMA for _ in range(NBUF)],
        ],
    )
    def emb_kernel(idx_hbm, table_hbm, out_hbm, idx_v, bufs, gsems, ssems):
        wid = lax.axis_index("s") * NC + lax.axis_index("c")
        i0 = wid * per_w

        pltpu.sync_copy(idx_hbm.at[pl.ds(i0, per_w)], idx_v)

        def fire_gather(g, b):
            pltpu.async_copy(table_hbm.at[idx_v.at[g]], bufs[b], gsems[b])

        def wait_gather(g, b):
            pltpu.make_async_copy(
                table_hbm.at[idx_v.at[g]], bufs[b], gsems[b]).wait()

        def fire_scatter(g, b):
            pltpu.async_copy(bufs[b], out_hbm.at[i0 + g], ssems[b])

        def wait_scatter(g, b):
            pltpu.make_async_copy(bufs[b], out_hbm.at[i0 + g], ssems[b]).wait()

        def scale_buf(b):
            buf = bufs[b]

            def row(r, carry):
                for c in range(D // LANES):
                    sl = (r, pl.ds(c * LANES, LANES))
                    buf[sl] = buf[sl] * scale
                return carry

            lax.fori_loop(0, hist, row, 0)

        for b in range(NBUF):
            fire_gather(b, b)

        def body(go, refire):
            for b in range(NBUF):
                g = go + b
                wait_gather(g, b)
                scale_buf(b)
                fire_scatter(g, b)
            if refire:
                for b in range(NBUF):
                    g = go + b
                    wait_scatter(g, b)
                    fire_gather(g + NBUF, b)

        @pl.loop(0, per_w - NBUF, step=NBUF)
        def _(go):
            body(go, refire=True)

        last = per_w - NBUF
        body(last, refire=False)
        for b in range(NBUF):
            wait_scatter(last + b, b)

    return emb_kernel


def kernel(indices, table):
    batch, hist = indices.shape
    return _build(batch, hist)(indices, table)


# speedup vs baseline: 5.2161x; 1.0002x over previous
import functools
import math

import jax
import jax.numpy as jnp
from jax import lax
from jax.experimental import pallas as pl
from jax.experimental.pallas import tpu as pltpu
from jax.experimental.pallas import tpu_sc as plsc

D = 128
LANES = 16
NC = 2
NS = 16
NW = NC * NS
NBUF = 8


def _build(batch, hist):
    per_w = batch // NW
    assert per_w * NW == batch and per_w % NBUF == 0
    scale = math.sqrt(float(D))

    mesh = plsc.VectorSubcoreMesh(core_axis_name="c", subcore_axis_name="s")

    @functools.partial(
        pl.kernel,
        out_type=jax.ShapeDtypeStruct((batch, hist, D), jnp.float32),
        mesh=mesh,
        compiler_params=pltpu.CompilerParams(use_tc_tiling_on_sc=True),
        scratch_types=[
            pltpu.VMEM((per_w, hist), jnp.int32),
            [pltpu.VMEM((hist, D), jnp.float32) for _ in range(NBUF)],
            [pltpu.SemaphoreType.DMA for _ in range(NBUF)],
            [pltpu.SemaphoreType.DMA for _ in range(NBUF)],
        ],
    )
    def emb_kernel(idx_hbm, table_hbm, out_hbm, idx_v, bufs, gsems, ssems):
        wid = lax.axis_index("s") * NC + lax.axis_index("c")
        i0 = wid * per_w

        pltpu.sync_copy(idx_hbm.at[pl.ds(i0, per_w)], idx_v)

        def fire_gather(g, b):
            pltpu.async_copy(table_hbm.at[idx_v.at[g]], bufs[b], gsems[b])

        def wait_gather(g, b):
            pltpu.make_async_copy(
                table_hbm.at[idx_v.at[g]], bufs[b], gsems[b]).wait()

        def fire_scatter(g, b):
            pltpu.async_copy(bufs[b], out_hbm.at[i0 + g], ssems[b])

        def wait_scatter(g, b):
            pltpu.make_async_copy(bufs[b], out_hbm.at[i0 + g], ssems[b]).wait()

        def scale_buf(b):
            buf = bufs[b]

            def row(r, carry):
                for c in range(D // LANES):
                    sl = (r, pl.ds(c * LANES, LANES))
                    buf[sl] = buf[sl] * scale
                return carry

            lax.fori_loop(0, hist, row, 0)

        for b in range(NBUF):
            fire_gather(b, b)

        def body(go, refire):
            for b in range(NBUF):
                g = go + b
                wait_gather(g, b)
                scale_buf(b)
                fire_scatter(g, b)
            if refire:
                for b in range(NBUF):
                    g = go + b
                    wait_scatter(g, b)
                    fire_gather(g + NBUF, b)

        @pl.loop(0, per_w - NBUF, step=NBUF)
        def _(go):
            body(go, refire=True)

        last = per_w - NBUF
        body(last, refire=False)
        for b in range(NBUF):
            wait_scatter(last + b, b)

    return emb_kernel


def kernel(indices, table):
    batch, hist = indices.shape
    return _build(batch, hist)(indices, table)


# speedup vs baseline: 9.1547x; 1.7551x over previous
import functools
import math

import jax
import jax.numpy as jnp
from jax import lax
from jax.experimental import pallas as pl
from jax.experimental.pallas import tpu as pltpu
from jax.experimental.pallas import tpu_sc as plsc

D = 128
LANES = 16
NC = 2
NS = 16
NW = NC * NS
NBUF = 5


def _build(batch, hist):
    per_w = batch // NW
    assert per_w * NW == batch and per_w <= 128
    assert hist % NBUF == 0
    scale = math.sqrt(float(D))

    mesh = plsc.VectorSubcoreMesh(core_axis_name="c", subcore_axis_name="s")

    @functools.partial(
        pl.kernel,
        out_type=jax.ShapeDtypeStruct((hist, batch, D), jnp.float32),
        mesh=mesh,
        scratch_types=[
            pltpu.VMEM((hist, per_w), jnp.int32),
            [pltpu.VMEM((per_w, D), jnp.float32) for _ in range(NBUF)],
            [pltpu.SemaphoreType.DMA for _ in range(NBUF)],
            [pltpu.SemaphoreType.DMA for _ in range(NBUF)],
        ],
    )
    def emb_kernel(idx_hbm, table_hbm, out_hbm, idx_v, bufs, gsems, ssems):
        wid = lax.axis_index("s") * NC + lax.axis_index("c")
        i0 = wid * per_w

        pltpu.sync_copy(idx_hbm.at[:, pl.ds(i0, per_w)], idx_v)

        def fire_gather(g, b):
            pltpu.async_copy(table_hbm.at[idx_v.at[g]], bufs[b], gsems[b])

        def wait_gather(g, b):
            pltpu.make_async_copy(
                table_hbm.at[idx_v.at[g]], bufs[b], gsems[b]).wait()

        def out_slice(g):
            return out_hbm.at[g, pl.ds(i0, per_w)]

        def fire_scatter(g, b):
            pltpu.async_copy(bufs[b], out_slice(g), ssems[b])

        def wait_scatter(g, b):
            pltpu.make_async_copy(bufs[b], out_slice(g), ssems[b]).wait()

        def scale_buf(b):
            buf = bufs[b]

            def row(r, carry):
                for c in range(D // LANES):
                    sl = (r, pl.ds(c * LANES, LANES))
                    buf[sl] = buf[sl] * scale
                return carry

            lax.fori_loop(0, per_w, row, 0)

        for b in range(NBUF):
            fire_gather(b, b)

        def body(go, refire):
            for b in range(NBUF):
                g = go + b
                wait_gather(g, b)
                scale_buf(b)
                fire_scatter(g, b)
            if refire:
                for b in range(NBUF):
                    g = go + b
                    wait_scatter(g, b)
                    fire_gather(g + NBUF, b)

        @pl.loop(0, hist - NBUF, step=NBUF)
        def _(go):
            body(go, refire=True)

        last = hist - NBUF
        body(last, refire=False)
        for b in range(NBUF):
            wait_scatter(last + b, b)

    return emb_kernel


def kernel(indices, table):
    batch, hist = indices.shape
    out = _build(batch, hist)(indices.T, table)
    return out.transpose(1, 0, 2)
